# baseline (speedup 1.0000x reference)
import jax
import jax.numpy as jnp
from jax import lax
from jax.experimental import pallas as pl
from jax.experimental.pallas import tpu as pltpu
from jax.experimental.pallas import tpu_sc as plsc

B, H, W, L = 8, 90, 256, 256
WL = W * L
HT = 54
NC, NS, LANES = 2, 16, 16
NW = NC * NS
NQ = NW // B
CH = WL // NQ
NV = CH // LANES
HB = 18
NH = HT // HB


def _tc_body(cs_ref, csp_ref, m_ref, out_ref, acc_ref):
    b = pl.program_id(0)
    h = pl.program_id(1)

    @pl.when((b == 0) & (h == 0))
    def _init():
        acc_ref[...] = jnp.zeros_like(acc_ref)

    a = cs_ref[0]
    p = csp_ref[0]
    m = m_ref[0, 0]
    a = jnp.where(jnp.isnan(a), 0.0, a)
    p = jnp.where(m > 0.0, p, 0.0)
    d = a - p
    acc_ref[...] += jnp.sum(d * d, axis=0)

    @pl.when((b == B - 1) & (h == NH - 1))
    def _fin():
        out_ref[0, 0] = jnp.sum(acc_ref[...])


def _tc_call(cs4, cs_p, m4):
    return pl.pallas_call(
        _tc_body,
        grid=(B, NH),
        in_specs=[
            pl.BlockSpec((1, HB, W, L), lambda b, h: (b, h, 0, 0)),
            pl.BlockSpec((1, HB, W, L), lambda b, h: (b, h, 0, 0)),
            pl.BlockSpec((1, 1, W, L), lambda b, h: (b, 0, 0, 0)),
        ],
        out_specs=pl.BlockSpec(memory_space=pltpu.SMEM),
        out_shape=jax.ShapeDtypeStruct((1, 1), jnp.float32),
        scratch_shapes=[pltpu.VMEM((W, L), jnp.float32)],
    )(cs4, cs_p, m4)


def _sc_body(cs_hbm, csp_hbm, m_hbm, out_hbm,
             mbuf, a0, a1, p0, p1, obuf, sa0, sp0, sa1, sp1):
    c = lax.axis_index("c")
    s = lax.axis_index("s")
    wid = s * NC + c
    b = wid // NQ
    qoff = (wid % NQ) * CH

    pltpu.sync_copy(m_hbm.at[b, pl.ds(qoff, CH)], mbuf)

    def _binm(i, carry):
        m = mbuf[pl.ds(i * LANES, LANES)]
        mbuf[pl.ds(i * LANES, LANES)] = jnp.where(m > 0.0, 1.0, 0.0)
        return carry
    lax.fori_loop(0, NV, _binm, 0)

    pltpu.async_copy(cs_hbm.at[b, HT, pl.ds(qoff, CH)], a0, sa0)
    pltpu.async_copy(csp_hbm.at[b, HT, pl.ds(qoff, CH)], p0, sp0)
    pltpu.async_copy(cs_hbm.at[b, HT + 1, pl.ds(qoff, CH)], a1, sa1)
    pltpu.async_copy(csp_hbm.at[b, HT + 1, pl.ds(qoff, CH)], p1, sp1)

    def _chunk(abuf, pbuf, acc):
        def _inner(i, acc):
            a = abuf[pl.ds(i * LANES, LANES)]
            p = pbuf[pl.ds(i * LANES, LANES)]
            m = mbuf[pl.ds(i * LANES, LANES)]
            a = jnp.where(jnp.isnan(a), 0.0, a)
            d = a - p * m
            return acc + d * d
        return lax.fori_loop(0, NV, _inner, acc, unroll=8)

    def _outer(k, acc):
        h0 = HT + 2 * k
        pltpu.make_async_copy(cs_hbm.at[b, h0, pl.ds(qoff, CH)], a0, sa0).wait()
        pltpu.make_async_copy(csp_hbm.at[b, h0, pl.ds(qoff, CH)], p0, sp0).wait()
        acc = _chunk(a0, p0, acc)

        @pl.when(h0 + 2 < H)
        def _():
            pltpu.async_copy(cs_hbm.at[b, h0 + 2, pl.ds(qoff, CH)], a0, sa0)
            pltpu.async_copy(csp_hbm.at[b, h0 + 2, pl.ds(qoff, CH)], p0, sp0)

        pltpu.make_async_copy(cs_hbm.at[b, h0 + 1, pl.ds(qoff, CH)], a1, sa1).wait()
        pltpu.make_async_copy(csp_hbm.at[b, h0 + 1, pl.ds(qoff, CH)], p1, sp1).wait()
        acc = _chunk(a1, p1, acc)

        @pl.when(h0 + 3 < H)
        def _():
            pltpu.async_copy(cs_hbm.at[b, h0 + 3, pl.ds(qoff, CH)], a1, sa1)
            pltpu.async_copy(csp_hbm.at[b, h0 + 3, pl.ds(qoff, CH)], p1, sp1)
        return acc

    acc = lax.fori_loop(0, (H - HT) // 2, _outer, jnp.zeros((LANES,), jnp.float32))
    obuf[...] = acc
    pltpu.sync_copy(obuf, out_hbm.at[wid])


_mesh = plsc.VectorSubcoreMesh(core_axis_name="c", subcore_axis_name="s")

_sc_call = pl.kernel(
    _sc_body,
    out_type=jax.ShapeDtypeStruct((NW, LANES), jnp.float32),
    mesh=_mesh,
    scratch_types=[
        pltpu.VMEM((CH,), jnp.float32),
        pltpu.VMEM((CH,), jnp.float32),
        pltpu.VMEM((CH,), jnp.float32),
        pltpu.VMEM((CH,), jnp.float32),
        pltpu.VMEM((CH,), jnp.float32),
        pltpu.VMEM((LANES,), jnp.float32),
        pltpu.SemaphoreType.DMA,
        pltpu.SemaphoreType.DMA,
        pltpu.SemaphoreType.DMA,
        pltpu.SemaphoreType.DMA,
    ],
)


def kernel(cs, cs_p, overpass_mask):
    cs4 = cs.reshape(B, H, W, L)
    csp3 = cs_p.reshape(B, H, WL)
    m2 = overpass_mask.reshape(B, WL)
    m4 = overpass_mask.reshape(B, 1, W, L)
    sc_partials = _sc_call(cs, csp3, m2)
    tc_sum = _tc_call(cs4, cs_p, m4)
    total = tc_sum[0, 0] + jnp.sum(sc_partials)
    return total / jnp.float32(B * H * WL)


# speedup vs baseline: 1.9415x; 1.9415x over previous
import jax
import jax.numpy as jnp
from jax import lax
from jax.experimental import pallas as pl
from jax.experimental.pallas import tpu as pltpu
from jax.experimental.pallas import tpu_sc as plsc

B, H, W, L = 8, 90, 256, 256
WL = W * L
NC, NS, LANES = 2, 16, 16
NW = NC * NS
NQ = NW // B
CH = WL // NQ
WQ = W // NQ
NV = CH // LANES


def _sc_body(cs_hbm, csp_hbm, m_hbm, out_hbm,
             mbuf, a0, a1, p0, p1, obuf, sa0, sp0, sa1, sp1):
    c = lax.axis_index("c")
    s = lax.axis_index("s")
    wid = s * NC + c
    b = wid // NQ
    q = wid % NQ
    qoff = q * CH
    w0 = q * WQ

    pltpu.sync_copy(m_hbm.at[b, pl.ds(w0, WQ), :], mbuf)

    def _binm(i, carry):
        w = i >> 4
        g = i & 15
        m = mbuf[w, pl.ds(g * LANES, LANES)]
        mbuf[w, pl.ds(g * LANES, LANES)] = jnp.where(m > 0.0, 1.0, 0.0)
        return carry
    lax.fori_loop(0, NV, _binm, 0)

    pltpu.async_copy(cs_hbm.at[b, 0, pl.ds(qoff, CH)], a0, sa0)
    pltpu.async_copy(csp_hbm.at[b, 0, pl.ds(w0, WQ), :], p0, sp0)
    pltpu.async_copy(cs_hbm.at[b, 1, pl.ds(qoff, CH)], a1, sa1)
    pltpu.async_copy(csp_hbm.at[b, 1, pl.ds(w0, WQ), :], p1, sp1)

    def _chunk(abuf, pbuf, acc):
        def _inner(i, acc):
            w = i >> 4
            g = i & 15
            a = abuf[pl.ds(i * LANES, LANES)]
            p = pbuf[w, pl.ds(g * LANES, LANES)]
            m = mbuf[w, pl.ds(g * LANES, LANES)]
            a = jnp.where(jnp.isnan(a), 0.0, a)
            d = a - p * m
            return acc + d * d
        return lax.fori_loop(0, NV, _inner, acc, unroll=8)

    def _outer(k, acc):
        h0 = 2 * k
        pltpu.make_async_copy(cs_hbm.at[b, h0, pl.ds(qoff, CH)], a0, sa0).wait()
        pltpu.make_async_copy(csp_hbm.at[b, h0, pl.ds(w0, WQ), :], p0, sp0).wait()
        acc = _chunk(a0, p0, acc)

        @pl.when(h0 + 2 < H)
        def _():
            pltpu.async_copy(cs_hbm.at[b, h0 + 2, pl.ds(qoff, CH)], a0, sa0)
            pltpu.async_copy(csp_hbm.at[b, h0 + 2, pl.ds(w0, WQ), :], p0, sp0)

        pltpu.make_async_copy(cs_hbm.at[b, h0 + 1, pl.ds(qoff, CH)], a1, sa1).wait()
        pltpu.make_async_copy(csp_hbm.at[b, h0 + 1, pl.ds(w0, WQ), :], p1, sp1).wait()
        acc = _chunk(a1, p1, acc)

        @pl.when(h0 + 3 < H)
        def _():
            pltpu.async_copy(cs_hbm.at[b, h0 + 3, pl.ds(qoff, CH)], a1, sa1)
            pltpu.async_copy(csp_hbm.at[b, h0 + 3, pl.ds(w0, WQ), :], p1, sp1)
        return acc

    acc = lax.fori_loop(0, H // 2, _outer, jnp.zeros((LANES,), jnp.float32))
    obuf[...] = acc
    pltpu.sync_copy(obuf, out_hbm.at[wid])


_mesh = plsc.VectorSubcoreMesh(core_axis_name="c", subcore_axis_name="s")

_sc_call = pl.kernel(
    _sc_body,
    out_type=jax.ShapeDtypeStruct((NW, LANES), jnp.float32),
    mesh=_mesh,
    scratch_types=[
        pltpu.VMEM((WQ, L), jnp.float32),
        pltpu.VMEM((CH,), jnp.float32),
        pltpu.VMEM((CH,), jnp.float32),
        pltpu.VMEM((WQ, L), jnp.float32),
        pltpu.VMEM((WQ, L), jnp.float32),
        pltpu.VMEM((LANES,), jnp.float32),
        pltpu.SemaphoreType.DMA,
        pltpu.SemaphoreType.DMA,
        pltpu.SemaphoreType.DMA,
        pltpu.SemaphoreType.DMA,
    ],
)


def kernel(cs, cs_p, overpass_mask):
    partials = _sc_call(cs, cs_p, overpass_mask)
    return jnp.sum(partials) / jnp.float32(B * H * WL)


# speedup vs baseline: 2.3166x; 1.1932x over previous
import jax
import jax.numpy as jnp
from jax import lax
from jax.experimental import pallas as pl
from jax.experimental.pallas import tpu as pltpu
from jax.experimental.pallas import tpu_sc as plsc

B, H, W, L = 8, 90, 256, 256
WL = W * L
HS = 48
NC, NS, LANES = 2, 16, 16
NW = NC * NS
NQ = NW // B
CH = WL // NQ
WQ = W // NQ
NV = CH // LANES
HBT = 16
NHT = HS // HBT


def _tc_body(cs_ref, csp_ref, m_ref, out_ref, acc_ref):
    b = pl.program_id(0)
    k = pl.program_id(1)

    @pl.when((b == 0) & (k == 0))
    def _init():
        acc_ref[...] = jnp.zeros_like(acc_ref)

    vacc = jnp.zeros((HBT, L), jnp.float32)
    for w in range(W):
        a = cs_ref[0, :, pl.ds(w * L, L)]
        p = csp_ref[0, :, w, :]
        m = m_ref[0, w, :]
        a = jnp.where(jnp.isnan(a), 0.0, a)
        d = a - jnp.where(m > 0.0, p, 0.0)
        vacc = vacc + d * d
    acc_ref[...] += vacc

    @pl.when((b == B - 1) & (k == NHT - 1))
    def _fin():
        out_ref[0, 0] = jnp.sum(acc_ref[...])


def _tc_call(cs, cs_p, m):
    return pl.pallas_call(
        _tc_body,
        grid=(B, NHT),
        in_specs=[
            pl.BlockSpec((1, HBT, WL), lambda b, k: (b, k, 0)),
            pl.BlockSpec((1, HBT, W, L), lambda b, k: (b, k, 0, 0)),
            pl.BlockSpec((1, W, L), lambda b, k: (b, 0, 0)),
        ],
        out_specs=pl.BlockSpec(memory_space=pltpu.SMEM),
        out_shape=jax.ShapeDtypeStruct((1, 1), jnp.float32),
        scratch_shapes=[pltpu.VMEM((HBT, L), jnp.float32)],
    )(cs, cs_p, m)


def _sc_body(cs_hbm, csp_hbm, m_hbm, out_hbm,
             mbuf, a0, a1, p0, p1, obuf, sa0, sp0, sa1, sp1):
    c = lax.axis_index("c")
    s = lax.axis_index("s")
    wid = s * NC + c
    b = wid // NQ
    q = wid % NQ
    qoff = q * CH
    w0 = q * WQ

    pltpu.sync_copy(m_hbm.at[b, pl.ds(w0, WQ), :], mbuf)

    def _binm(i, carry):
        w = i >> 4
        g = i & 15
        m = mbuf[w, pl.ds(g * LANES, LANES)]
        mbuf[w, pl.ds(g * LANES, LANES)] = jnp.where(m > 0.0, 1.0, 0.0)
        return carry
    lax.fori_loop(0, NV, _binm, 0)

    pltpu.async_copy(cs_hbm.at[b, HS, pl.ds(qoff, CH)], a0, sa0)
    pltpu.async_copy(csp_hbm.at[b, HS, pl.ds(w0, WQ), :], p0, sp0)
    pltpu.async_copy(cs_hbm.at[b, HS + 1, pl.ds(qoff, CH)], a1, sa1)
    pltpu.async_copy(csp_hbm.at[b, HS + 1, pl.ds(w0, WQ), :], p1, sp1)

    def _chunk(abuf, pbuf, acc):
        def _inner(i, acc):
            w = i >> 4
            g = i & 15
            a = abuf[pl.ds(i * LANES, LANES)]
            p = pbuf[w, pl.ds(g * LANES, LANES)]
            m = mbuf[w, pl.ds(g * LANES, LANES)]
            a = jnp.where(jnp.isnan(a), 0.0, a)
            d = a - p * m
            return acc + d * d
        return lax.fori_loop(0, NV, _inner, acc, unroll=8)

    def _outer(k, acc):
        h0 = HS + 2 * k
        pltpu.make_async_copy(cs_hbm.at[b, h0, pl.ds(qoff, CH)], a0, sa0).wait()
        pltpu.make_async_copy(csp_hbm.at[b, h0, pl.ds(w0, WQ), :], p0, sp0).wait()
        acc = _chunk(a0, p0, acc)

        @pl.when(h0 + 2 < H)
        def _():
            pltpu.async_copy(cs_hbm.at[b, h0 + 2, pl.ds(qoff, CH)], a0, sa0)
            pltpu.async_copy(csp_hbm.at[b, h0 + 2, pl.ds(w0, WQ), :], p0, sp0)

        pltpu.make_async_copy(cs_hbm.at[b, h0 + 1, pl.ds(qoff, CH)], a1, sa1).wait()
        pltpu.make_async_copy(csp_hbm.at[b, h0 + 1, pl.ds(w0, WQ), :], p1, sp1).wait()
        acc = _chunk(a1, p1, acc)

        @pl.when(h0 + 3 < H)
        def _():
            pltpu.async_copy(cs_hbm.at[b, h0 + 3, pl.ds(qoff, CH)], a1, sa1)
            pltpu.async_copy(csp_hbm.at[b, h0 + 3, pl.ds(w0, WQ), :], p1, sp1)
        return acc

    acc = lax.fori_loop(0, (H - HS) // 2, _outer, jnp.zeros((LANES,), jnp.float32))
    obuf[...] = acc
    pltpu.sync_copy(obuf, out_hbm.at[wid])


_mesh = plsc.VectorSubcoreMesh(core_axis_name="c", subcore_axis_name="s")

_sc_call = pl.kernel(
    _sc_body,
    out_type=jax.ShapeDtypeStruct((NW, LANES), jnp.float32),
    mesh=_mesh,
    scratch_types=[
        pltpu.VMEM((WQ, L), jnp.float32),
        pltpu.VMEM((CH,), jnp.float32),
        pltpu.VMEM((CH,), jnp.float32),
        pltpu.VMEM((WQ, L), jnp.float32),
        pltpu.VMEM((WQ, L), jnp.float32),
        pltpu.VMEM((LANES,), jnp.float32),
        pltpu.SemaphoreType.DMA,
        pltpu.SemaphoreType.DMA,
        pltpu.SemaphoreType.DMA,
        pltpu.SemaphoreType.DMA,
    ],
)


def kernel(cs, cs_p, overpass_mask):
    sc_partials = _sc_call(cs, cs_p, overpass_mask)
    tc_sum = _tc_call(cs, cs_p, overpass_mask)
    total = tc_sum[0, 0] + jnp.sum(sc_partials)
    return total / jnp.float32(B * H * WL)
